# Initial kernel scaffold; baseline (speedup 1.0000x reference)
import jax
import jax.numpy as jnp
from jax.experimental import pallas as pl


def kernel(x, edge_index, W_in, b_in, W1, b1, g1, be1, W2, b2, g2, be2, W3, b3, g3, be3, Wc, bc):
    raise NotImplementedError("write your pallas kernel here")


# speedup vs baseline: 10.4525x; 10.4525x over previous
import functools

import jax
import jax.numpy as jnp
from jax.experimental import pallas as pl
from jax.experimental.pallas import tpu as pltpu

BATCH = 256
N_NODES = 478
N_PAD = 480
E_PAD = 1024
D_IN_PAD = 8
D_H = 128
B_T = 8


def _adj_kernel(e_ref, a_ref):
    e = e_ref[...]
    src = e[0:1, :]
    dst = e[1:2, :]
    rows = jax.lax.broadcasted_iota(jnp.int32, (N_PAD, E_PAD), 0)
    oh_src = (src == rows).astype(jnp.float32)
    oh_dst = (dst == rows).astype(jnp.float32)
    counts = jax.lax.dot_general(
        oh_src, oh_dst, (((1,), (1,)), ((), ())),
        preferred_element_type=jnp.float32)
    adj = (counts > 0).astype(jnp.float32)
    deg = jnp.sum(adj, axis=1, keepdims=True) + 1e-6
    a_ref[...] = adj / deg


def _gnn_kernel(x_ref, a_ref, w_in_ref, b_in_ref,
                w1_ref, b1_ref, g1_ref, be1_ref,
                w2_ref, b2_ref, g2_ref, be2_ref,
                w3_ref, b3_ref, g3_ref, be3_ref,
                wc_ref, bc_ref, out_ref, agg_ref):
    R = B_T * N_PAD
    x = x_ref[...].reshape(R, D_IN_PAD)
    h = jnp.dot(x, w_in_ref[...], preferred_element_type=jnp.float32)
    h = h + b_in_ref[...]
    a = a_ref[...]
    layers = ((w1_ref, b1_ref, g1_ref, be1_ref),
              (w2_ref, b2_ref, g2_ref, be2_ref),
              (w3_ref, b3_ref, g3_ref, be3_ref))
    for w_ref, b_ref, g_ref, be_ref in layers:
        xt = jnp.dot(h, w_ref[...], preferred_element_type=jnp.float32)
        for bi in range(B_T):
            agg_ref[bi * N_PAD:(bi + 1) * N_PAD, :] = jnp.dot(
                a, xt[bi * N_PAD:(bi + 1) * N_PAD, :],
                preferred_element_type=jnp.float32)
        o = agg_ref[...] + b_ref[...]
        m = jnp.mean(o, axis=-1, keepdims=True)
        c = o - m
        v = jnp.mean(c * c, axis=-1, keepdims=True)
        o = c * jax.lax.rsqrt(v + 1e-5) * g_ref[...] + be_ref[...]
        o = 0.5 * o * (1.0 + jax.lax.erf(o * 0.7071067811865476))
        h = o + h
    ri = jax.lax.broadcasted_iota(jnp.int32, (B_T, R), 0)
    ci = jax.lax.broadcasted_iota(jnp.int32, (B_T, R), 1)
    node = ci - ri * N_PAD
    msk = ((node >= 0) & (node < N_NODES)).astype(jnp.float32) * (1.0 / N_NODES)
    pooled = jnp.dot(msk, h, preferred_element_type=jnp.float32)
    out_ref[...] = jnp.dot(pooled, wc_ref[...],
                           preferred_element_type=jnp.float32) + bc_ref[...]


@functools.partial(jax.jit, static_argnums=())
def kernel(x, edge_index, W_in, b_in, W1, b1, g1, be1, W2, b2, g2, be2,
           W3, b3, g3, be3, Wc, bc):
    f32 = jnp.float32
    n_classes = Wc.shape[1]
    e = jnp.full((8, E_PAD), -1, dtype=jnp.int32)
    e = e.at[:2, :edge_index.shape[1]].set(edge_index.astype(jnp.int32))
    x_pad = jnp.zeros((BATCH, N_PAD, D_IN_PAD), f32).at[:, :N_NODES, :x.shape[2]].set(x)
    w_in_pad = jnp.zeros((D_IN_PAD, D_H), f32).at[:W_in.shape[0], :].set(W_in)
    wc_pad = jnp.zeros((D_H, D_H), f32).at[:, :n_classes].set(Wc)
    bc_pad = jnp.zeros((1, D_H), f32).at[0, :n_classes].set(bc)
    row = lambda v: v.reshape(1, D_H)

    a_norm = pl.pallas_call(
        _adj_kernel,
        out_shape=jax.ShapeDtypeStruct((N_PAD, N_PAD), f32),
    )(e)

    grid = (BATCH // B_T,)
    full = lambda shape: pl.BlockSpec(shape, lambda i: (0,) * len(shape))
    out = pl.pallas_call(
        _gnn_kernel,
        grid=grid,
        in_specs=[
            pl.BlockSpec((B_T, N_PAD, D_IN_PAD), lambda i: (i, 0, 0)),
            full((N_PAD, N_PAD)),
            full((D_IN_PAD, D_H)), full((1, D_H)),
            full((D_H, D_H)), full((1, D_H)), full((1, D_H)), full((1, D_H)),
            full((D_H, D_H)), full((1, D_H)), full((1, D_H)), full((1, D_H)),
            full((D_H, D_H)), full((1, D_H)), full((1, D_H)), full((1, D_H)),
            full((D_H, D_H)), full((1, D_H)),
        ],
        out_specs=pl.BlockSpec((B_T, D_H), lambda i: (i, 0)),
        out_shape=jax.ShapeDtypeStruct((BATCH, D_H), f32),
        scratch_shapes=[pltpu.VMEM((B_T * N_PAD, D_H), f32)],
    )(x_pad, a_norm, w_in_pad, row(b_in),
      W1, row(b1), row(g1), row(be1),
      W2, row(b2), row(g2), row(be2),
      W3, row(b3), row(g3), row(be3),
      wc_pad, bc_pad)
    return out[:, :n_classes]
